# baseline (device time: 50588 ns/iter reference)
import jax
import jax.numpy as jnp
from jax import lax
from jax.experimental import pallas as pl
from jax.experimental.pallas import tpu as pltpu

C = 16
NQ = 4


def kernel(x):
    _, M, N2 = x.shape
    N = N2 // 2
    W = N // NQ
    Wh = W // 2
    R = M // C

    def body(x_ref, out_ref, res4, sbuf, rbuf, stage_p, stage_l,
             p_sems, l_sems, x_send, x_recv, zo_s, zo_r, yo_s, yo_r,
             zf_s, zf_r, yf_s, yf_r, o_sems):
        my_x = lax.axis_index("x")
        my_y = lax.axis_index("y")
        my_z = lax.axis_index("z")
        x_peer = (1 - my_x, my_y, my_z)
        z_peer = (my_x, my_y, 1 - my_z)
        y_peer = (my_x, 1 - my_y, my_z)
        my_q = 2 * my_y + my_z
        zq = 2 * my_y + (1 - my_z)
        yq = 2 * (1 - my_y) + my_z

        barrier = pltpu.get_barrier_semaphore()
        for nbr in (x_peer, z_peer, y_peer):
            pl.semaphore_signal(barrier, inc=1, device_id=nbr,
                                device_id_type=pl.DeviceIdType.MESH)
        pl.semaphore_wait(barrier, 3)

        cps_p, cps_l = [], []
        for c in range(C):
            rows = pl.ds(c * R, R)
            cp = pltpu.make_async_copy(
                x_ref.at[0, rows, pl.ds((1 - my_x) * N + my_q * W, W)],
                stage_p.at[rows, :], p_sems.at[c])
            cp.start()
            cps_p.append(cp)
            cp = pltpu.make_async_copy(
                x_ref.at[0, rows, pl.ds(my_x * N + my_q * W, W)],
                stage_l.at[rows, :], l_sems.at[c])
            cp.start()
            cps_l.append(cp)

        x_rdma = []
        for c in range(C):
            rows = pl.ds(c * R, R)
            cps_p[c].wait()
            sbuf[rows, :] = stage_p[rows, :].astype(jnp.bfloat16)
            r = pltpu.make_async_remote_copy(
                src_ref=sbuf.at[rows, :], dst_ref=rbuf.at[rows, :],
                send_sem=x_send.at[c], recv_sem=x_recv.at[c],
                device_id=x_peer, device_id_type=pl.DeviceIdType.MESH,
            )
            r.start()
            x_rdma.append(r)

        zo = [None] * C
        yo = [None] * C
        zf = [None] * C
        yf = [None] * C
        out_cps = []

        def start_own(c):
            rows = pl.ds(c * R, R)
            r = pltpu.make_async_remote_copy(
                src_ref=res4.at[my_q, rows, :],
                dst_ref=res4.at[my_q, rows, :],
                send_sem=zo_s.at[c], recv_sem=zo_r.at[c],
                device_id=z_peer, device_id_type=pl.DeviceIdType.MESH,
            )
            r.start()
            zo[c] = r
            r = pltpu.make_async_remote_copy(
                src_ref=res4.at[my_q, rows, :],
                dst_ref=res4.at[my_q, rows, :],
                send_sem=yo_s.at[c], recv_sem=yo_r.at[c],
                device_id=y_peer, device_id_type=pl.DeviceIdType.MESH,
            )
            r.start()
            yo[c] = r

        def start_yf(c):
            rows = pl.ds(c * R, R)
            r = pltpu.make_async_remote_copy(
                src_ref=res4.at[zq, rows, pl.ds(Wh, Wh)],
                dst_ref=res4.at[zq, rows, pl.ds(Wh, Wh)],
                send_sem=yf_s.at[c], recv_sem=yf_r.at[c],
                device_id=y_peer, device_id_type=pl.DeviceIdType.MESH,
            )
            r.start()
            yf[c] = r

        def start_zf(c):
            rows = pl.ds(c * R, R)
            r = pltpu.make_async_remote_copy(
                src_ref=res4.at[yq, rows, pl.ds(0, Wh)],
                dst_ref=res4.at[yq, rows, pl.ds(0, Wh)],
                send_sem=zf_s.at[c], recv_sem=zf_r.at[c],
                device_id=z_peer, device_id_type=pl.DeviceIdType.MESH,
            )
            r.start()
            zf[c] = r

        def store_out(c):
            rows = pl.ds(c * R, R)
            for q in range(NQ):
                cp = pltpu.make_async_copy(
                    res4.at[q, rows, :],
                    out_ref.at[rows, pl.ds(q * W, W)],
                    o_sems.at[c, q])
                cp.start()
                out_cps.append(cp)

        for c in range(C):
            rows = pl.ds(c * R, R)
            x_rdma[c].wait_recv()
            cps_l[c].wait()
            res4[my_q, rows, :] = (
                stage_l[rows, :] + rbuf[rows, :].astype(jnp.float32)
            ).astype(jnp.bfloat16)
            start_own(c)
            if c >= 1:
                zo[c - 1].wait_recv()
                start_yf(c - 1)
                yo[c - 1].wait_recv()
                start_zf(c - 1)
            if c >= 2:
                zf[c - 2].wait_recv()
                yf[c - 2].wait_recv()
                store_out(c - 2)

        zo[C - 1].wait_recv()
        start_yf(C - 1)
        yo[C - 1].wait_recv()
        start_zf(C - 1)
        for c in (C - 2, C - 1):
            zf[c].wait_recv()
            yf[c].wait_recv()
            store_out(c)

        for c in range(C):
            x_rdma[c].wait_send()
            zo[c].wait_send()
            yo[c].wait_send()
            zf[c].wait_send()
            yf[c].wait_send()
        for cp in out_cps:
            cp.wait()

    return pl.pallas_call(
        body,
        out_shape=jax.ShapeDtypeStruct((M, N), jnp.bfloat16),
        in_specs=[pl.BlockSpec(memory_space=pltpu.MemorySpace.HBM)],
        out_specs=pl.BlockSpec(memory_space=pltpu.MemorySpace.HBM),
        scratch_shapes=[
            pltpu.VMEM((NQ, M, W), jnp.bfloat16),
            pltpu.VMEM((M, W), jnp.bfloat16),
            pltpu.VMEM((M, W), jnp.bfloat16),
            pltpu.VMEM((M, W), jnp.float32),
            pltpu.VMEM((M, W), jnp.float32),
            pltpu.SemaphoreType.DMA((C,)),
            pltpu.SemaphoreType.DMA((C,)),
            pltpu.SemaphoreType.DMA((C,)),
            pltpu.SemaphoreType.DMA((C,)),
            pltpu.SemaphoreType.DMA((C,)),
            pltpu.SemaphoreType.DMA((C,)),
            pltpu.SemaphoreType.DMA((C,)),
            pltpu.SemaphoreType.DMA((C,)),
            pltpu.SemaphoreType.DMA((C,)),
            pltpu.SemaphoreType.DMA((C,)),
            pltpu.SemaphoreType.DMA((C,)),
            pltpu.SemaphoreType.DMA((C,)),
            pltpu.SemaphoreType.DMA((C, NQ)),
        ],
        compiler_params=pltpu.CompilerParams(collective_id=0),
    )(x)


# device time: 46192 ns/iter; 1.0952x vs baseline; 1.0952x over previous
import jax
import jax.numpy as jnp
from jax import lax
from jax.experimental import pallas as pl
from jax.experimental.pallas import tpu as pltpu

C = 8
FX = 3
NQ = 4


def kernel(x):
    _, M, N2 = x.shape
    N = N2 // 2
    W = N // NQ
    Wh = W // 2
    R = M // C

    def body(x_ref, out_ref, res4, sbuf, rbuf, sbuf2, rbuf2,
             stage_p, stage_l, stage_p2, stage_l2,
             p_sems, l_sems, p2_sems, l2_sems,
             x_send, x_recv, xd_s, xd_r, zo_s, zo_r, yo_s, yo_r,
             zf_s, zf_r, yf_s, yf_r, o_sems):
        my_x = lax.axis_index("x")
        my_y = lax.axis_index("y")
        my_z = lax.axis_index("z")
        x_peer = (1 - my_x, my_y, my_z)
        z_peer = (my_x, my_y, 1 - my_z)
        y_peer = (my_x, 1 - my_y, my_z)
        my_q = 2 * my_y + my_z
        zq = 2 * my_y + (1 - my_z)
        yq = 2 * (1 - my_y) + my_z
        dq = 2 * (1 - my_y) + (1 - my_z)

        barrier = pltpu.get_barrier_semaphore()
        for nbr in (x_peer, z_peer, y_peer):
            pl.semaphore_signal(barrier, inc=1, device_id=nbr,
                                device_id_type=pl.DeviceIdType.MESH)
        pl.semaphore_wait(barrier, 3)

        cps_p, cps_l, cps_p2, cps_l2 = [], [], [], []
        for c in range(C):
            rows = pl.ds(c * R, R)
            cp = pltpu.make_async_copy(
                x_ref.at[0, rows, pl.ds((1 - my_x) * N + my_q * W, W)],
                stage_p.at[rows, :], p_sems.at[c])
            cp.start()
            cps_p.append(cp)
            cp = pltpu.make_async_copy(
                x_ref.at[0, rows, pl.ds(my_x * N + my_q * W, W)],
                stage_l.at[rows, :], l_sems.at[c])
            cp.start()
            cps_l.append(cp)
        for c in range(FX):
            rows = pl.ds(c * R, R)
            cp = pltpu.make_async_copy(
                x_ref.at[0, rows, pl.ds((1 - my_x) * N + dq * W, W)],
                stage_p2.at[rows, :], p2_sems.at[c])
            cp.start()
            cps_p2.append(cp)
            cp = pltpu.make_async_copy(
                x_ref.at[0, rows, pl.ds(my_x * N + dq * W, W)],
                stage_l2.at[rows, :], l2_sems.at[c])
            cp.start()
            cps_l2.append(cp)

        x_rdma = []
        for c in range(C):
            rows = pl.ds(c * R, R)
            cps_p[c].wait()
            sbuf[rows, :] = stage_p[rows, :].astype(jnp.bfloat16)
            r = pltpu.make_async_remote_copy(
                src_ref=sbuf.at[rows, :], dst_ref=rbuf.at[rows, :],
                send_sem=x_send.at[c], recv_sem=x_recv.at[c],
                device_id=x_peer, device_id_type=pl.DeviceIdType.MESH,
            )
            r.start()
            x_rdma.append(r)
        xd_rdma = []
        for c in range(FX):
            rows = pl.ds(c * R, R)
            cps_p2[c].wait()
            sbuf2[rows, :] = stage_p2[rows, :].astype(jnp.bfloat16)
            r = pltpu.make_async_remote_copy(
                src_ref=sbuf2.at[rows, :], dst_ref=rbuf2.at[rows, :],
                send_sem=xd_s.at[c], recv_sem=xd_r.at[c],
                device_id=x_peer, device_id_type=pl.DeviceIdType.MESH,
            )
            r.start()
            xd_rdma.append(r)

        zo = [None] * C
        yo = [None] * C
        zf = [None] * C
        yf = [None] * C
        out_cps = []

        def start_own(c):
            rows = pl.ds(c * R, R)
            r = pltpu.make_async_remote_copy(
                src_ref=res4.at[my_q, rows, :],
                dst_ref=res4.at[my_q, rows, :],
                send_sem=zo_s.at[c], recv_sem=zo_r.at[c],
                device_id=z_peer, device_id_type=pl.DeviceIdType.MESH,
            )
            r.start()
            zo[c] = r
            r = pltpu.make_async_remote_copy(
                src_ref=res4.at[my_q, rows, :],
                dst_ref=res4.at[my_q, rows, :],
                send_sem=yo_s.at[c], recv_sem=yo_r.at[c],
                device_id=y_peer, device_id_type=pl.DeviceIdType.MESH,
            )
            r.start()
            yo[c] = r

        def start_yf(c):
            rows = pl.ds(c * R, R)
            r = pltpu.make_async_remote_copy(
                src_ref=res4.at[zq, rows, pl.ds(Wh, Wh)],
                dst_ref=res4.at[zq, rows, pl.ds(Wh, Wh)],
                send_sem=yf_s.at[c], recv_sem=yf_r.at[c],
                device_id=y_peer, device_id_type=pl.DeviceIdType.MESH,
            )
            r.start()
            yf[c] = r

        def start_zf(c):
            rows = pl.ds(c * R, R)
            r = pltpu.make_async_remote_copy(
                src_ref=res4.at[yq, rows, pl.ds(0, Wh)],
                dst_ref=res4.at[yq, rows, pl.ds(0, Wh)],
                send_sem=zf_s.at[c], recv_sem=zf_r.at[c],
                device_id=z_peer, device_id_type=pl.DeviceIdType.MESH,
            )
            r.start()
            zf[c] = r

        def store_out(c):
            rows = pl.ds(c * R, R)
            for q in range(NQ):
                cp = pltpu.make_async_copy(
                    res4.at[q, rows, :],
                    out_ref.at[rows, pl.ds(q * W, W)],
                    o_sems.at[c, q])
                cp.start()
                out_cps.append(cp)

        def finish_fwd(k):
            zf[k].wait_recv()
            yf[k].wait_recv()
            store_out(k)

        for c in range(C):
            rows = pl.ds(c * R, R)
            x_rdma[c].wait_recv()
            cps_l[c].wait()
            res4[my_q, rows, :] = (
                stage_l[rows, :] + rbuf[rows, :].astype(jnp.float32)
            ).astype(jnp.bfloat16)
            start_own(c)
            if c >= 1:
                k = c - 1
                zo[k].wait_recv()
                yo[k].wait_recv()
                if k >= FX:
                    start_yf(k)
                    start_zf(k)
            if c >= 2 and c - 2 >= FX:
                finish_fwd(c - 2)

        zo[C - 1].wait_recv()
        yo[C - 1].wait_recv()
        start_yf(C - 1)
        start_zf(C - 1)
        finish_fwd(C - 2)
        finish_fwd(C - 1)

        for c in range(FX):
            rows = pl.ds(c * R, R)
            xd_rdma[c].wait_recv()
            cps_l2[c].wait()
            res4[dq, rows, :] = (
                stage_l2[rows, :] + rbuf2[rows, :].astype(jnp.float32)
            ).astype(jnp.bfloat16)
            store_out(c)

        for c in range(C):
            x_rdma[c].wait_send()
            zo[c].wait_send()
            yo[c].wait_send()
            if c >= FX:
                zf[c].wait_send()
                yf[c].wait_send()
        for c in range(FX):
            xd_rdma[c].wait_send()
        for cp in out_cps:
            cp.wait()

    return pl.pallas_call(
        body,
        out_shape=jax.ShapeDtypeStruct((M, N), jnp.bfloat16),
        in_specs=[pl.BlockSpec(memory_space=pltpu.MemorySpace.HBM)],
        out_specs=pl.BlockSpec(memory_space=pltpu.MemorySpace.HBM),
        scratch_shapes=[
            pltpu.VMEM((NQ, M, W), jnp.bfloat16),
            pltpu.VMEM((M, W), jnp.bfloat16),
            pltpu.VMEM((M, W), jnp.bfloat16),
            pltpu.VMEM((FX * R, W), jnp.bfloat16),
            pltpu.VMEM((FX * R, W), jnp.bfloat16),
            pltpu.VMEM((M, W), jnp.float32),
            pltpu.VMEM((M, W), jnp.float32),
            pltpu.VMEM((FX * R, W), jnp.float32),
            pltpu.VMEM((FX * R, W), jnp.float32),
            pltpu.SemaphoreType.DMA((C,)),
            pltpu.SemaphoreType.DMA((C,)),
            pltpu.SemaphoreType.DMA((FX,)),
            pltpu.SemaphoreType.DMA((FX,)),
            pltpu.SemaphoreType.DMA((C,)),
            pltpu.SemaphoreType.DMA((C,)),
            pltpu.SemaphoreType.DMA((FX,)),
            pltpu.SemaphoreType.DMA((FX,)),
            pltpu.SemaphoreType.DMA((C,)),
            pltpu.SemaphoreType.DMA((C,)),
            pltpu.SemaphoreType.DMA((C,)),
            pltpu.SemaphoreType.DMA((C,)),
            pltpu.SemaphoreType.DMA((C,)),
            pltpu.SemaphoreType.DMA((C,)),
            pltpu.SemaphoreType.DMA((C,)),
            pltpu.SemaphoreType.DMA((C,)),
            pltpu.SemaphoreType.DMA((C, NQ)),
        ],
        compiler_params=pltpu.CompilerParams(collective_id=0),
    )(x)


# device time: 45146 ns/iter; 1.1205x vs baseline; 1.0232x over previous
import jax
import jax.numpy as jnp
from jax import lax
from jax.experimental import pallas as pl
from jax.experimental.pallas import tpu as pltpu

C = 8
FX = 3
NQ = 4


def kernel(x):
    _, M, N2 = x.shape
    N = N2 // 2
    W = N // NQ
    Wh = W // 2
    R = M // C

    def body(x_ref, out_ref, res4, sbuf, rbuf, sbuf2, rbuf2,
             stage_p, stage_l, stage_p2, stage_l2,
             p_sems, l_sems, p2_sems, l2_sems,
             x_send, x_recv, xd_s, xd_r, zo_s, zo_r, yo_s, yo_r,
             zf_s, zf_r, yf_s, yf_r, o_sems):
        my_x = lax.axis_index("x")
        my_y = lax.axis_index("y")
        my_z = lax.axis_index("z")
        x_peer = (1 - my_x, my_y, my_z)
        z_peer = (my_x, my_y, 1 - my_z)
        y_peer = (my_x, 1 - my_y, my_z)
        my_q = 2 * my_y + my_z
        zq = 2 * my_y + (1 - my_z)
        yq = 2 * (1 - my_y) + my_z
        dq = 2 * (1 - my_y) + (1 - my_z)

        barrier = pltpu.get_barrier_semaphore()
        for nbr in (x_peer, z_peer, y_peer):
            pl.semaphore_signal(barrier, inc=1, device_id=nbr,
                                device_id_type=pl.DeviceIdType.MESH)
        pl.semaphore_wait(barrier, 3)

        cps_p, cps_l, cps_p2, cps_l2 = [], [], [], []
        for c in range(C):
            rows = pl.ds(c * R, R)
            cp = pltpu.make_async_copy(
                x_ref.at[0, rows, pl.ds((1 - my_x) * N + my_q * W, W)],
                stage_p.at[rows, :], p_sems.at[c])
            cp.start()
            cps_p.append(cp)
            cp = pltpu.make_async_copy(
                x_ref.at[0, rows, pl.ds(my_x * N + my_q * W, W)],
                stage_l.at[rows, :], l_sems.at[c])
            cp.start()
            cps_l.append(cp)
        for c in range(FX):
            rows = pl.ds(c * R, R)
            cp = pltpu.make_async_copy(
                x_ref.at[0, rows, pl.ds((1 - my_x) * N + dq * W, W)],
                stage_p2.at[rows, :], p2_sems.at[c])
            cp.start()
            cps_p2.append(cp)
            cp = pltpu.make_async_copy(
                x_ref.at[0, rows, pl.ds(my_x * N + dq * W, W)],
                stage_l2.at[rows, :], l2_sems.at[c])
            cp.start()
            cps_l2.append(cp)

        x_rdma = []
        for c in range(C):
            rows = pl.ds(c * R, R)
            cps_p[c].wait()
            sbuf[rows, :] = stage_p[rows, :].astype(jnp.bfloat16)
            r = pltpu.make_async_remote_copy(
                src_ref=sbuf.at[rows, :], dst_ref=rbuf.at[rows, :],
                send_sem=x_send.at[c], recv_sem=x_recv.at[c],
                device_id=x_peer, device_id_type=pl.DeviceIdType.MESH,
            )
            r.start()
            x_rdma.append(r)
        xd_rdma = []
        for c in range(FX):
            rows = pl.ds(c * R, R)
            cps_p2[c].wait()
            sbuf2[rows, :] = stage_p2[rows, :].astype(jnp.bfloat16)
            r = pltpu.make_async_remote_copy(
                src_ref=sbuf2.at[rows, :], dst_ref=rbuf2.at[rows, :],
                send_sem=xd_s.at[c], recv_sem=xd_r.at[c],
                device_id=x_peer, device_id_type=pl.DeviceIdType.MESH,
            )
            r.start()
            xd_rdma.append(r)

        zo = [None] * C
        yo = [None] * C
        zf = [None] * C
        yf = [None] * C
        out_cps = []

        def start_own(c):
            rows = pl.ds(c * R, R)
            r = pltpu.make_async_remote_copy(
                src_ref=res4.at[my_q, rows, :],
                dst_ref=res4.at[my_q, rows, :],
                send_sem=zo_s.at[c], recv_sem=zo_r.at[c],
                device_id=z_peer, device_id_type=pl.DeviceIdType.MESH,
            )
            r.start()
            zo[c] = r
            r = pltpu.make_async_remote_copy(
                src_ref=res4.at[my_q, rows, :],
                dst_ref=res4.at[my_q, rows, :],
                send_sem=yo_s.at[c], recv_sem=yo_r.at[c],
                device_id=y_peer, device_id_type=pl.DeviceIdType.MESH,
            )
            r.start()
            yo[c] = r

        def start_yf(c):
            rows = pl.ds(c * R, R)
            r = pltpu.make_async_remote_copy(
                src_ref=res4.at[zq, rows, pl.ds(Wh, Wh)],
                dst_ref=res4.at[zq, rows, pl.ds(Wh, Wh)],
                send_sem=yf_s.at[c], recv_sem=yf_r.at[c],
                device_id=y_peer, device_id_type=pl.DeviceIdType.MESH,
            )
            r.start()
            yf[c] = r

        def start_zf(c):
            rows = pl.ds(c * R, R)
            r = pltpu.make_async_remote_copy(
                src_ref=res4.at[yq, rows, pl.ds(0, Wh)],
                dst_ref=res4.at[yq, rows, pl.ds(0, Wh)],
                send_sem=zf_s.at[c], recv_sem=zf_r.at[c],
                device_id=z_peer, device_id_type=pl.DeviceIdType.MESH,
            )
            r.start()
            zf[c] = r

        def store_out(c):
            rows = pl.ds(c * R, R)
            for q in range(NQ):
                cp = pltpu.make_async_copy(
                    res4.at[q, rows, :],
                    out_ref.at[rows, pl.ds(q * W, W)],
                    o_sems.at[c, q])
                cp.start()
                out_cps.append(cp)

        def finish_fwd(k):
            zf[k].wait_recv()
            yf[k].wait_recv()
            store_out(k)

        for c in range(C):
            rows = pl.ds(c * R, R)
            x_rdma[c].wait_recv()
            cps_l[c].wait()
            res4[my_q, rows, :] = (
                stage_l[rows, :] + rbuf[rows, :].astype(jnp.float32)
            ).astype(jnp.bfloat16)
            start_own(c)
            if c >= 1:
                k = c - 1
                zo[k].wait_recv()
                yo[k].wait_recv()
                if k >= FX:
                    start_yf(k)
                    start_zf(k)
            if c >= 2 and c - 2 >= FX:
                finish_fwd(c - 2)

        zo[C - 1].wait_recv()
        yo[C - 1].wait_recv()
        start_yf(C - 1)
        start_zf(C - 1)

        for c in range(FX):
            rows = pl.ds(c * R, R)
            xd_rdma[c].wait_recv()
            cps_l2[c].wait()
            res4[dq, rows, :] = (
                stage_l2[rows, :] + rbuf2[rows, :].astype(jnp.float32)
            ).astype(jnp.bfloat16)
            store_out(c)

        finish_fwd(C - 2)
        finish_fwd(C - 1)

        for c in range(C):
            x_rdma[c].wait_send()
            zo[c].wait_send()
            yo[c].wait_send()
            if c >= FX:
                zf[c].wait_send()
                yf[c].wait_send()
        for c in range(FX):
            xd_rdma[c].wait_send()
        for cp in out_cps:
            cp.wait()

    return pl.pallas_call(
        body,
        out_shape=jax.ShapeDtypeStruct((M, N), jnp.bfloat16),
        in_specs=[pl.BlockSpec(memory_space=pltpu.MemorySpace.HBM)],
        out_specs=pl.BlockSpec(memory_space=pltpu.MemorySpace.HBM),
        scratch_shapes=[
            pltpu.VMEM((NQ, M, W), jnp.bfloat16),
            pltpu.VMEM((M, W), jnp.bfloat16),
            pltpu.VMEM((M, W), jnp.bfloat16),
            pltpu.VMEM((FX * R, W), jnp.bfloat16),
            pltpu.VMEM((FX * R, W), jnp.bfloat16),
            pltpu.VMEM((M, W), jnp.float32),
            pltpu.VMEM((M, W), jnp.float32),
            pltpu.VMEM((FX * R, W), jnp.float32),
            pltpu.VMEM((FX * R, W), jnp.float32),
            pltpu.SemaphoreType.DMA((C,)),
            pltpu.SemaphoreType.DMA((C,)),
            pltpu.SemaphoreType.DMA((FX,)),
            pltpu.SemaphoreType.DMA((FX,)),
            pltpu.SemaphoreType.DMA((C,)),
            pltpu.SemaphoreType.DMA((C,)),
            pltpu.SemaphoreType.DMA((FX,)),
            pltpu.SemaphoreType.DMA((FX,)),
            pltpu.SemaphoreType.DMA((C,)),
            pltpu.SemaphoreType.DMA((C,)),
            pltpu.SemaphoreType.DMA((C,)),
            pltpu.SemaphoreType.DMA((C,)),
            pltpu.SemaphoreType.DMA((C,)),
            pltpu.SemaphoreType.DMA((C,)),
            pltpu.SemaphoreType.DMA((C,)),
            pltpu.SemaphoreType.DMA((C,)),
            pltpu.SemaphoreType.DMA((C, NQ)),
        ],
        compiler_params=pltpu.CompilerParams(collective_id=0),
    )(x)
